# baseline (speedup 1.0000x reference)
import functools

import jax
import jax.numpy as jnp
from jax import lax
from jax.experimental import pallas as pl
from jax.experimental.pallas import tpu as pltpu
from jax.experimental.pallas import tpu_sc as plsc

NC = 2
NS = 16
NW = NC * NS
FL = 16

M = 256


def _sc_mesh():
    return plsc.VectorSubcoreMesh(
        core_axis_name="c", subcore_axis_name="s", num_cores=NC, num_subcores=NS
    )


def _wid():
    return lax.axis_index("s") * NC + lax.axis_index("c")


def _gather_rows(P, D, C):
    NBUF = 3

    @functools.partial(
        pl.kernel,
        out_type=jax.ShapeDtypeStruct((P, D), jnp.float32),
        mesh=_sc_mesh(),
        scratch_types=[
            pltpu.VMEM((P // NW,), jnp.int32),
        ]
        + [pltpu.VMEM((C, D), jnp.float32) for _ in range(NBUF)]
        + [pltpu.SemaphoreType.DMA for _ in range(2 * NBUF)],
    )
    def gather_k(tok_hbm, x_hbm, out_hbm, idx_v, *bufs_sems):
        bufs = bufs_sems[:NBUF]
        sgs = bufs_sems[NBUF : 2 * NBUF]
        sws = bufs_sems[2 * NBUF :]
        rows_per_w = P // NW
        base = _wid() * rows_per_w
        n = rows_per_w // C

        pltpu.sync_copy(tok_hbm.at[pl.ds(base, rows_per_w)], idx_v)
        g = [None] * NBUF
        wb = [None] * NBUF

        def drain(cd):
            sd = cd % NBUF
            g[sd].wait()
            wb[sd] = pltpu.async_copy(
                bufs[sd], out_hbm.at[pl.ds(base + cd * C, C)], sws[sd]
            )

        for c in range(n):
            s = c % NBUF
            if wb[s] is not None:
                wb[s].wait()
                wb[s] = None
            g[s] = pltpu.async_copy(
                x_hbm.at[idx_v.at[pl.ds(c * C, C)]], bufs[s], sgs[s]
            )
            if c >= NBUF - 1:
                drain(c - (NBUF - 1))
        for cd in range(max(0, n - (NBUF - 1)), n):
            drain(cd)
        for w in wb:
            if w is not None:
                w.wait()

    return gather_k


def _combine_rows(T, D, K, Ct):
    assert K == 2

    @functools.partial(
        pl.kernel,
        out_type=jax.ShapeDtypeStruct((T, D), jnp.float32),
        mesh=_sc_mesh(),
        scratch_types=[
            pltpu.VMEM((T // NW,), jnp.int32),
            pltpu.VMEM((T // NW,), jnp.int32),
            pltpu.VMEM((Ct, D), jnp.float32),
            pltpu.VMEM((Ct, D), jnp.float32),
            pltpu.VMEM((Ct, D), jnp.float32),
            pltpu.VMEM((Ct, D), jnp.float32),
            pltpu.SemaphoreType.DMA,
            pltpu.SemaphoreType.DMA,
            pltpu.SemaphoreType.DMA,
            pltpu.SemaphoreType.DMA,
            pltpu.SemaphoreType.DMA,
            pltpu.SemaphoreType.DMA,
        ],
    )
    def combine_k(pos_hbm, yw_hbm, out_hbm, i0_v, i1_v,
                  a0, a1, b0, b1, sa0, sa1, sb0, sb1, sw0, sw1):
        tok_per_w = T // NW
        base = _wid() * tok_per_w
        n = tok_per_w // Ct
        A = (a0, a1)
        B = (b0, b1)
        sA = (sa0, sa1)
        sB = (sb0, sb1)
        sW = (sw0, sw1)
        nvec = D // FL

        pltpu.sync_copy(pos_hbm.at[0, pl.ds(base, tok_per_w)], i0_v)
        pltpu.sync_copy(pos_hbm.at[1, pl.ds(base, tok_per_w)], i1_v)

        def issue(c, s):
            return (
                pltpu.async_copy(yw_hbm.at[i0_v.at[pl.ds(c * Ct, Ct)]], A[s], sA[s]),
                pltpu.async_copy(yw_hbm.at[i1_v.at[pl.ds(c * Ct, Ct)]], B[s], sB[s]),
            )

        g = [None, None]
        wb = [None, None]
        g[0] = issue(0, 0)
        for c in range(n):
            a = c % 2
            b = 1 - a
            g[a][0].wait()
            g[a][1].wait()
            if c + 1 < n:
                if wb[b] is not None:
                    wb[b].wait()
                    wb[b] = None
                g[b] = issue(c + 1, b)

            UN = 8

            def add_row(r, carry):
                def add_vec(j, c3):
                    for u in range(UN):
                        sl = pl.ds((j * UN + u) * FL, FL)
                        A[a][r, sl] = A[a][r, sl] + B[a][r, sl]
                    return c3

                lax.fori_loop(0, nvec // UN, add_vec, 0)
                return carry

            lax.fori_loop(0, Ct, add_row, 0)
            wb[a] = pltpu.async_copy(
                A[a], out_hbm.at[pl.ds(base + c * Ct, Ct)], sW[a]
            )
        for w in wb:
            if w is not None:
                w.wait()

    return combine_k


def _grouped_mlp(NT, D, F):
    def body(te_ref, tv_ref, xg_ref, w1_ref, w2_ref, rw_ref, out_ref):
        j = pl.program_id(0)

        @pl.when(tv_ref[j] == 1)
        def _():
            xb = xg_ref[...].astype(jnp.bfloat16)
            w1 = w1_ref[0].astype(jnp.bfloat16)
            h = jnp.dot(xb, w1, preferred_element_type=jnp.float32)
            hb = jnp.maximum(h, 0.0).astype(jnp.bfloat16)
            w2 = w2_ref[0].astype(jnp.bfloat16)
            y = jnp.dot(hb, w2, preferred_element_type=jnp.float32)
            out_ref[...] = y * rw_ref[0, 0, :][:, None]

    grid_spec = pltpu.PrefetchScalarGridSpec(
        num_scalar_prefetch=2,
        grid=(NT,),
        in_specs=[
            pl.BlockSpec((M, D), lambda j, te, tv: (j, 0)),
            pl.BlockSpec((1, D, F), lambda j, te, tv: (te[j], 0, 0)),
            pl.BlockSpec((1, F, D), lambda j, te, tv: (te[j], 0, 0)),
            pl.BlockSpec((1, 1, M), lambda j, te, tv: (j, 0, 0)),
        ],
        out_specs=pl.BlockSpec((M, D), lambda j, te, tv: (j, 0)),
    )
    return pl.pallas_call(
        body,
        grid_spec=grid_spec,
        out_shape=jax.ShapeDtypeStruct((NT * M, D), jnp.float32),
        compiler_params=pltpu.CompilerParams(
            dimension_semantics=("arbitrary",),
        ),
    )


def kernel(x, topk_indices, topk_weights, num_experts, W1, W2):
    T, D = x.shape
    E, _, F = W1.shape
    K = topk_indices.shape[1]
    N = T * K
    NT = (N + E * M) // M
    P = NT * M

    e_flat = topk_indices.reshape(N).astype(jnp.int32)
    w_flat = topk_weights.reshape(N) * (e_flat < num_experts).astype(jnp.float32)
    oh = (e_flat[:, None] == jnp.arange(E, dtype=jnp.int32)[None, :]).astype(jnp.int32)
    ranks_incl = jnp.cumsum(oh, axis=0)
    rank = jnp.sum(ranks_incl * oh, axis=1) - 1
    counts = ranks_incl[-1]
    padded = ((counts + M - 1) // M) * M
    pad_start = jnp.concatenate(
        [jnp.zeros(1, jnp.int32), jnp.cumsum(padded).astype(jnp.int32)]
    )
    dest = pad_start[e_flat] + rank
    total = pad_start[E]

    row_token = (jnp.arange(P, dtype=jnp.int32) % T).at[dest].set(
        jnp.arange(N, dtype=jnp.int32) // K
    )
    row_weight = jnp.zeros(P, jnp.float32).at[dest].set(w_flat)
    pos_kt = dest.reshape(T, K).T

    tile_starts = jnp.arange(NT, dtype=jnp.int32) * M
    tile_expert = jnp.clip(
        jnp.searchsorted(pad_start[1:], tile_starts, side="right"), 0, E - 1
    ).astype(jnp.int32)
    tile_valid = (tile_starts < total).astype(jnp.int32)

    xg = _gather_rows(P, D, 32)(row_token, x)
    yw = _grouped_mlp(NT, D, F)(
        tile_expert, tile_valid, xg, W1, W2, row_weight.reshape(NT, 1, M)
    )
    return _combine_rows(T, D, K, 16)(pos_kt, yw)


# speedup vs baseline: 1.1276x; 1.1276x over previous
import functools

import jax
import jax.numpy as jnp
from jax import lax
from jax.experimental import pallas as pl
from jax.experimental.pallas import tpu as pltpu
from jax.experimental.pallas import tpu_sc as plsc

NC = 2
NS = 16
NW = NC * NS
FL = 16

M = 256


def _sc_mesh():
    return plsc.VectorSubcoreMesh(
        core_axis_name="c", subcore_axis_name="s", num_cores=NC, num_subcores=NS
    )


def _wid():
    return lax.axis_index("s") * NC + lax.axis_index("c")


def _dispatch_rows(T, D, K, P, C):
    assert K == 2

    @functools.partial(
        pl.kernel,
        out_type=(
            jax.ShapeDtypeStruct((P, D), jnp.float32),
            jax.ShapeDtypeStruct((P,), jnp.float32),
        ),
        mesh=_sc_mesh(),
        scratch_types=(
            [pltpu.VMEM((C, D), jnp.float32) for _ in range(2)]
            + [pltpu.VMEM((C,), jnp.int32) for _ in range(4)]
            + [pltpu.VMEM((C,), jnp.float32) for _ in range(4)]
            + [pltpu.SemaphoreType.DMA for _ in range(5)]
        ),
    )
    def dispatch_k(x_hbm, dkt_hbm, wkt_hbm, xg_hbm, rw_hbm, *scr):
        rows = scr[0:2]
        i0 = scr[2:4]
        i1 = scr[4:6]
        w0 = scr[6:8]
        w1 = scr[8:10]
        sl, s0, s1, sw0, sw1 = scr[10:15]
        tok_per_w = T // NW
        base = _wid() * tok_per_w
        n = tok_per_w // C

        ld = [None, None]
        ld[0] = pltpu.async_copy(x_hbm.at[pl.ds(base, C)], rows[0], sl)
        pending = []
        for c in range(n):
            a = c % 2
            tok0 = base + c * C
            pltpu.sync_copy(dkt_hbm.at[0, pl.ds(tok0, C)], i0[a])
            pltpu.sync_copy(dkt_hbm.at[1, pl.ds(tok0, C)], i1[a])
            pltpu.sync_copy(wkt_hbm.at[0, pl.ds(tok0, C)], w0[a])
            pltpu.sync_copy(wkt_hbm.at[1, pl.ds(tok0, C)], w1[a])
            ld[a].wait()
            for p in pending:
                p.wait()
            if c + 1 < n:
                ld[1 - a] = pltpu.async_copy(
                    x_hbm.at[pl.ds(tok0 + C, C)], rows[1 - a], sl
                )
            pending = [
                pltpu.async_copy(rows[a], xg_hbm.at[i0[a]], s0),
                pltpu.async_copy(rows[a], xg_hbm.at[i1[a]], s1),
                pltpu.async_copy(w0[a], rw_hbm.at[i0[a]], sw0),
                pltpu.async_copy(w1[a], rw_hbm.at[i1[a]], sw1),
            ]
        for p in pending:
            p.wait()

    return dispatch_k


def _combine_rows(T, D, K, Ct):
    assert K == 2

    @functools.partial(
        pl.kernel,
        out_type=jax.ShapeDtypeStruct((T, D), jnp.float32),
        mesh=_sc_mesh(),
        scratch_types=[
            pltpu.VMEM((T // NW,), jnp.int32),
            pltpu.VMEM((T // NW,), jnp.int32),
            pltpu.VMEM((Ct, D), jnp.float32),
            pltpu.VMEM((Ct, D), jnp.float32),
            pltpu.VMEM((Ct, D), jnp.float32),
            pltpu.VMEM((Ct, D), jnp.float32),
            pltpu.SemaphoreType.DMA,
            pltpu.SemaphoreType.DMA,
            pltpu.SemaphoreType.DMA,
            pltpu.SemaphoreType.DMA,
            pltpu.SemaphoreType.DMA,
            pltpu.SemaphoreType.DMA,
        ],
    )
    def combine_k(pos_hbm, yw_hbm, out_hbm, i0_v, i1_v,
                  a0, a1, b0, b1, sa0, sa1, sb0, sb1, sw0, sw1):
        tok_per_w = T // NW
        base = _wid() * tok_per_w
        n = tok_per_w // Ct
        A = (a0, a1)
        B = (b0, b1)
        sA = (sa0, sa1)
        sB = (sb0, sb1)
        sW = (sw0, sw1)
        nvec = D // FL

        pltpu.sync_copy(pos_hbm.at[0, pl.ds(base, tok_per_w)], i0_v)
        pltpu.sync_copy(pos_hbm.at[1, pl.ds(base, tok_per_w)], i1_v)

        def issue(c, s):
            return (
                pltpu.async_copy(yw_hbm.at[i0_v.at[pl.ds(c * Ct, Ct)]], A[s], sA[s]),
                pltpu.async_copy(yw_hbm.at[i1_v.at[pl.ds(c * Ct, Ct)]], B[s], sB[s]),
            )

        g = [None, None]
        wb = [None, None]
        g[0] = issue(0, 0)
        for c in range(n):
            a = c % 2
            b = 1 - a
            g[a][0].wait()
            g[a][1].wait()
            if c + 1 < n:
                if wb[b] is not None:
                    wb[b].wait()
                    wb[b] = None
                g[b] = issue(c + 1, b)

            UN = 8

            def add_row(r, carry):
                def add_vec(j, c3):
                    for u in range(UN):
                        sl = pl.ds((j * UN + u) * FL, FL)
                        A[a][r, sl] = A[a][r, sl] + B[a][r, sl]
                    return c3

                lax.fori_loop(0, nvec // UN, add_vec, 0)
                return carry

            lax.fori_loop(0, Ct, add_row, 0)
            wb[a] = pltpu.async_copy(
                A[a], out_hbm.at[pl.ds(base + c * Ct, Ct)], sW[a]
            )
        for w in wb:
            if w is not None:
                w.wait()

    return combine_k


def _grouped_mlp(NT, D, F):
    def body(te_ref, tv_ref, xg_ref, w1_ref, w2_ref, rw_ref, out_ref):
        j = pl.program_id(0)

        @pl.when(tv_ref[j] == 1)
        def _():
            xb = xg_ref[...].astype(jnp.bfloat16)
            w1 = w1_ref[0].astype(jnp.bfloat16)
            h = jnp.dot(xb, w1, preferred_element_type=jnp.float32)
            hb = jnp.maximum(h, 0.0).astype(jnp.bfloat16)
            w2 = w2_ref[0].astype(jnp.bfloat16)
            y = jnp.dot(hb, w2, preferred_element_type=jnp.float32)
            out_ref[...] = y * rw_ref[0, 0, :][:, None]

    grid_spec = pltpu.PrefetchScalarGridSpec(
        num_scalar_prefetch=2,
        grid=(NT,),
        in_specs=[
            pl.BlockSpec((M, D), lambda j, te, tv: (j, 0)),
            pl.BlockSpec((1, D, F), lambda j, te, tv: (te[j], 0, 0)),
            pl.BlockSpec((1, F, D), lambda j, te, tv: (te[j], 0, 0)),
            pl.BlockSpec((1, 1, M), lambda j, te, tv: (j, 0, 0)),
        ],
        out_specs=pl.BlockSpec((M, D), lambda j, te, tv: (j, 0)),
    )
    return pl.pallas_call(
        body,
        grid_spec=grid_spec,
        out_shape=jax.ShapeDtypeStruct((NT * M, D), jnp.float32),
        compiler_params=pltpu.CompilerParams(
            dimension_semantics=("arbitrary",),
        ),
    )


def kernel(x, topk_indices, topk_weights, num_experts, W1, W2):
    T, D = x.shape
    E, _, F = W1.shape
    K = topk_indices.shape[1]
    N = T * K
    NT = (N + E * M) // M
    P = NT * M

    BS = 128
    NB = N // BS
    e_flat = topk_indices.reshape(N).astype(jnp.int32)
    w_flat = topk_weights.reshape(N) * (e_flat < num_experts).astype(jnp.float32)
    ohf = (e_flat[:, None] == jnp.arange(E, dtype=jnp.int32)[None, :]).astype(
        jnp.float32
    )
    tri = jnp.tril(jnp.ones((BS, BS), jnp.float32))
    within = jax.lax.dot_general(
        tri, ohf.reshape(NB, BS, E),
        dimension_numbers=(((1,), (1,)), ((), ())),
        preferred_element_type=jnp.float32,
    )
    within = jnp.transpose(within, (1, 0, 2))
    bsum = within[:, -1, :]
    tri_x = jnp.tril(jnp.ones((NB, NB), jnp.float32), k=-1)
    boff = jax.lax.dot_general(
        tri_x, bsum,
        dimension_numbers=(((1,), (0,)), ((), ())),
        preferred_element_type=jnp.float32,
    )
    ranks_incl = (within + boff[:, None, :]).reshape(N, E)
    rank = (jnp.sum(ranks_incl * ohf, axis=1) - 1.0).astype(jnp.int32)
    counts = jnp.sum(bsum, axis=0).astype(jnp.int32)
    padded = ((counts + M - 1) // M) * M
    pad_start = jnp.concatenate(
        [jnp.zeros(1, jnp.int32), jnp.cumsum(padded).astype(jnp.int32)]
    )
    dest = pad_start[e_flat] + rank
    total = pad_start[E]

    dest_kt = dest.reshape(T, K).T
    w_kt = w_flat.reshape(T, K).T

    tile_starts = jnp.arange(NT, dtype=jnp.int32) * M
    tile_expert = jnp.clip(
        jnp.sum((tile_starts[:, None] >= pad_start[None, 1:]).astype(jnp.int32),
                axis=1),
        0, E - 1,
    )
    tile_valid = (tile_starts < total).astype(jnp.int32)

    xg, rw = _dispatch_rows(T, D, K, P, 32)(x, dest_kt, w_kt)
    yw = _grouped_mlp(NT, D, F)(
        tile_expert, tile_valid, xg, W1, W2, rw.reshape(NT, 1, M)
    )
    return _combine_rows(T, D, K, 16)(dest_kt, yw)


# speedup vs baseline: 1.1659x; 1.0340x over previous
import functools

import jax
import jax.numpy as jnp
from jax import lax
from jax.experimental import pallas as pl
from jax.experimental.pallas import tpu as pltpu
from jax.experimental.pallas import tpu_sc as plsc

NC = 2
NS = 16
NW = NC * NS
FL = 16

M = 256


def _sc_mesh():
    return plsc.VectorSubcoreMesh(
        core_axis_name="c", subcore_axis_name="s", num_cores=NC, num_subcores=NS
    )


def _wid():
    return lax.axis_index("s") * NC + lax.axis_index("c")


def _dispatch_rows(T, D, K, P, C):
    assert K == 2

    @functools.partial(
        pl.kernel,
        out_type=(
            jax.ShapeDtypeStruct((P, D), jnp.float32),
            jax.ShapeDtypeStruct((P,), jnp.float32),
        ),
        mesh=_sc_mesh(),
        scratch_types=(
            [pltpu.VMEM((C, D), jnp.float32) for _ in range(2)]
            + [pltpu.VMEM((C,), jnp.int32) for _ in range(4)]
            + [pltpu.VMEM((T // NW,), jnp.int32) for _ in range(2)]
            + [pltpu.VMEM((T // NW,), jnp.float32) for _ in range(2)]
            + [pltpu.SemaphoreType.DMA for _ in range(5)]
        ),
    )
    def dispatch_k(x_hbm, dkt_hbm, wkt_hbm, xg_hbm, rw_hbm, *scr):
        rows = scr[0:2]
        i0 = scr[2:4]
        i1 = scr[4:6]
        iw0, iw1, w0, w1 = scr[6:10]
        sl, s0, s1, sw0, sw1 = scr[10:15]
        tok_per_w = T // NW
        base = _wid() * tok_per_w
        n = tok_per_w // C

        pltpu.sync_copy(dkt_hbm.at[0, pl.ds(base, tok_per_w)], iw0)
        pltpu.sync_copy(dkt_hbm.at[1, pl.ds(base, tok_per_w)], iw1)
        pltpu.sync_copy(wkt_hbm.at[0, pl.ds(base, tok_per_w)], w0)
        pltpu.sync_copy(wkt_hbm.at[1, pl.ds(base, tok_per_w)], w1)
        wsc = [
            pltpu.async_copy(w0, rw_hbm.at[iw0], sw0),
            pltpu.async_copy(w1, rw_hbm.at[iw1], sw1),
        ]

        ld = [None, None]
        ld[0] = pltpu.async_copy(x_hbm.at[pl.ds(base, C)], rows[0], sl)
        pending = []
        for c in range(n):
            a = c % 2
            tok0 = base + c * C
            pltpu.sync_copy(dkt_hbm.at[0, pl.ds(tok0, C)], i0[a])
            pltpu.sync_copy(dkt_hbm.at[1, pl.ds(tok0, C)], i1[a])
            ld[a].wait()
            for p in pending:
                p.wait()
            if c + 1 < n:
                ld[1 - a] = pltpu.async_copy(
                    x_hbm.at[pl.ds(tok0 + C, C)], rows[1 - a], sl
                )
            pending = [
                pltpu.async_copy(rows[a], xg_hbm.at[i0[a]], s0),
                pltpu.async_copy(rows[a], xg_hbm.at[i1[a]], s1),
            ]
        for p in pending:
            p.wait()
        for p in wsc:
            p.wait()

    return dispatch_k


def _combine_rows(T, D, K, Ct):
    assert K == 2

    @functools.partial(
        pl.kernel,
        out_type=jax.ShapeDtypeStruct((T, D), jnp.float32),
        mesh=_sc_mesh(),
        scratch_types=[
            pltpu.VMEM((T // NW,), jnp.int32),
            pltpu.VMEM((T // NW,), jnp.int32),
            pltpu.VMEM((Ct, D), jnp.float32),
            pltpu.VMEM((Ct, D), jnp.float32),
            pltpu.VMEM((Ct, D), jnp.float32),
            pltpu.VMEM((Ct, D), jnp.float32),
            pltpu.SemaphoreType.DMA,
            pltpu.SemaphoreType.DMA,
            pltpu.SemaphoreType.DMA,
            pltpu.SemaphoreType.DMA,
            pltpu.SemaphoreType.DMA,
            pltpu.SemaphoreType.DMA,
        ],
    )
    def combine_k(pos_hbm, yw_hbm, out_hbm, i0_v, i1_v,
                  a0, a1, b0, b1, sa0, sa1, sb0, sb1, sw0, sw1):
        tok_per_w = T // NW
        base = _wid() * tok_per_w
        n = tok_per_w // Ct
        A = (a0, a1)
        B = (b0, b1)
        sA = (sa0, sa1)
        sB = (sb0, sb1)
        sW = (sw0, sw1)
        nvec = D // FL

        pltpu.sync_copy(pos_hbm.at[0, pl.ds(base, tok_per_w)], i0_v)
        pltpu.sync_copy(pos_hbm.at[1, pl.ds(base, tok_per_w)], i1_v)

        def issue(c, s):
            return (
                pltpu.async_copy(yw_hbm.at[i0_v.at[pl.ds(c * Ct, Ct)]], A[s], sA[s]),
                pltpu.async_copy(yw_hbm.at[i1_v.at[pl.ds(c * Ct, Ct)]], B[s], sB[s]),
            )

        g = [None, None]
        wb = [None, None]
        g[0] = issue(0, 0)
        for c in range(n):
            a = c % 2
            b = 1 - a
            g[a][0].wait()
            g[a][1].wait()
            if c + 1 < n:
                if wb[b] is not None:
                    wb[b].wait()
                    wb[b] = None
                g[b] = issue(c + 1, b)

            def add_row(r, carry):
                def add_vec(j, c3):
                    sl = pl.ds(j * FL, FL)
                    A[a][r, sl] = A[a][r, sl] + B[a][r, sl]
                    return c3

                lax.fori_loop(0, nvec, add_vec, 0)
                return carry

            lax.fori_loop(0, Ct, add_row, 0)
            wb[a] = pltpu.async_copy(
                A[a], out_hbm.at[pl.ds(base + c * Ct, Ct)], sW[a]
            )
        for w in wb:
            if w is not None:
                w.wait()

    return combine_k


def _grouped_mlp(NT, D, F):
    def body(te_ref, tv_ref, xg_ref, w1_ref, w2_ref, rw_ref, out_ref):
        j = pl.program_id(0)

        @pl.when(tv_ref[j] == 1)
        def _():
            xb = xg_ref[...].astype(jnp.bfloat16)
            w1 = w1_ref[0].astype(jnp.bfloat16)
            h = jnp.dot(xb, w1, preferred_element_type=jnp.float32)
            hb = jnp.maximum(h, 0.0).astype(jnp.bfloat16)
            w2 = w2_ref[0].astype(jnp.bfloat16)
            y = jnp.dot(hb, w2, preferred_element_type=jnp.float32)
            out_ref[...] = y * rw_ref[0, 0, :][:, None]

    grid_spec = pltpu.PrefetchScalarGridSpec(
        num_scalar_prefetch=2,
        grid=(NT,),
        in_specs=[
            pl.BlockSpec((M, D), lambda j, te, tv: (j, 0)),
            pl.BlockSpec((1, D, F), lambda j, te, tv: (te[j], 0, 0)),
            pl.BlockSpec((1, F, D), lambda j, te, tv: (te[j], 0, 0)),
            pl.BlockSpec((1, 1, M), lambda j, te, tv: (j, 0, 0)),
        ],
        out_specs=pl.BlockSpec((M, D), lambda j, te, tv: (j, 0)),
    )
    return pl.pallas_call(
        body,
        grid_spec=grid_spec,
        out_shape=jax.ShapeDtypeStruct((NT * M, D), jnp.float32),
        compiler_params=pltpu.CompilerParams(
            dimension_semantics=("arbitrary",),
        ),
    )


def kernel(x, topk_indices, topk_weights, num_experts, W1, W2):
    T, D = x.shape
    E, _, F = W1.shape
    K = topk_indices.shape[1]
    N = T * K
    NT = (N + E * M) // M
    P = NT * M

    BS = 128
    NB = N // BS
    e_flat = topk_indices.reshape(N).astype(jnp.int32)
    w_flat = topk_weights.reshape(N) * (e_flat < num_experts).astype(jnp.float32)
    ohf = (e_flat[:, None] == jnp.arange(E, dtype=jnp.int32)[None, :]).astype(
        jnp.float32
    )
    tri = jnp.tril(jnp.ones((BS, BS), jnp.float32))
    within = jax.lax.dot_general(
        tri, ohf.reshape(NB, BS, E),
        dimension_numbers=(((1,), (1,)), ((), ())),
        preferred_element_type=jnp.float32,
    )
    within = jnp.transpose(within, (1, 0, 2))
    bsum = within[:, -1, :]
    tri_x = jnp.tril(jnp.ones((NB, NB), jnp.float32), k=-1)
    boff = jax.lax.dot_general(
        tri_x, bsum,
        dimension_numbers=(((1,), (0,)), ((), ())),
        preferred_element_type=jnp.float32,
    )
    ranks_incl = (within + boff[:, None, :]).reshape(N, E)
    rank = (jnp.sum(ranks_incl * ohf, axis=1) - 1.0).astype(jnp.int32)
    counts = jnp.sum(bsum, axis=0).astype(jnp.int32)
    padded = ((counts + M - 1) // M) * M
    pad_start = jnp.concatenate(
        [jnp.zeros(1, jnp.int32), jnp.cumsum(padded).astype(jnp.int32)]
    )
    dest = pad_start[e_flat] + rank
    total = pad_start[E]

    dest_kt = dest.reshape(T, K).T
    w_kt = w_flat.reshape(T, K).T

    tile_starts = jnp.arange(NT, dtype=jnp.int32) * M
    tile_expert = jnp.clip(
        jnp.sum((tile_starts[:, None] >= pad_start[None, 1:]).astype(jnp.int32),
                axis=1),
        0, E - 1,
    )
    tile_valid = (tile_starts < total).astype(jnp.int32)

    xg, rw = _dispatch_rows(T, D, K, P, 32)(x, dest_kt, w_kt)
    yw = _grouped_mlp(NT, D, F)(
        tile_expert, tile_valid, xg, W1, W2, rw.reshape(NT, 1, M)
    )
    return _combine_rows(T, D, K, 16)(dest_kt, yw)
